# baseline (speedup 1.0000x reference)
import numpy as np
import jax
import jax.numpy as jnp
from jax import lax
from jax.experimental import pallas as pl
from jax.experimental.pallas import tpu as pltpu
from jax.experimental.pallas import tpu_sc as plsc

B, A, C, M = 4, 100000, 4, 32
NW = 32
LANES = 16
HA = A // 2
AW = 1600
STEPS = AW // LANES
LAST_START = HA - AW
LAST_DUP_STEPS = (30 * AW + AW - LAST_START) // LANES

ALPHA = np.float32(0.95)
ONE_MINUS_ALPHA = np.float32(0.05)
DAMPENING = np.float32(0.5)
T_NEG = np.float32(900.0)
T_POS = np.float32(399.99996948242188)
MAX_ANGLE_DIST = np.float32(30.0)
NEG_ANGLE_DIST = np.float32(45.0)
LN2 = np.float32(0.6931471805599453)


def _splat_i32(v):
    return jnp.full((LANES,), v, jnp.int32)


def _soft_log(v):
    bits = lax.bitcast_convert_type(v, jnp.int32)
    e = (bits >> 23) - 127
    mbits = (bits & jnp.int32(0x7FFFFF)) | jnp.int32(0x3F800000)
    m = lax.bitcast_convert_type(mbits, jnp.float32)
    ef = e.astype(jnp.float32)
    big = m > jnp.float32(1.4142135)
    m = jnp.where(big, m * jnp.float32(0.5), m)
    ef = jnp.where(big, ef + jnp.float32(1.0), ef)
    t = (m - jnp.float32(1.0)) / (m + jnp.float32(1.0))
    t2 = t * t
    p = jnp.float32(1.0 / 7.0)
    p = p * t2 + jnp.float32(1.0 / 5.0)
    p = p * t2 + jnp.float32(1.0 / 3.0)
    p = p * t2 + jnp.float32(1.0)
    return ef * LN2 + jnp.float32(2.0) * t * p


def _soft_sqrt(x):
    bits = lax.bitcast_convert_type(x, jnp.int32)
    y = lax.bitcast_convert_type(jnp.int32(0x5F3759DF) - (bits >> 1), jnp.float32)
    xh = jnp.float32(0.5) * x
    for _ in range(3):
        y = y * (jnp.float32(1.5) - xh * y * y)
    return x * y


def _smooth_l1(d):
    return jnp.where(d <= jnp.float32(1.0 / 9.0),
                     jnp.float32(4.5) * d * d,
                     d - jnp.float32(1.0 / 18.0))


def _focal_body(cls_hbm, reg_hbm, anch_hbm, annb_hbm, annt_hbm, st_hbm,
                out_hbm, cls_v, reg_v, st_v, anch_v, annb_v, annt_v, out_v,
                sem):
    wid = lax.axis_index("s") * 2 + lax.axis_index("c")
    start = jnp.minimum(wid * AW, LAST_START)

    copies = [
        pltpu.make_async_copy(annb_hbm, annb_v, sem),
        pltpu.make_async_copy(annt_hbm, annt_v, sem),
    ]
    for k in range(3):
        copies.append(pltpu.make_async_copy(
            anch_hbm.at[pl.ds(k * HA + start, AW)],
            anch_v.at[pl.ds(k * AW, AW)], sem))
    for b in range(B):
        for c in range(C):
            copies.append(pltpu.make_async_copy(
                cls_hbm.at[pl.ds((b * C + c) * HA + start, AW)],
                cls_v.at[pl.ds((b * C + c) * AW, AW)], sem))
        for r in range(3):
            copies.append(pltpu.make_async_copy(
                reg_hbm.at[pl.ds((b * 3 + r) * HA + start, AW)],
                reg_v.at[pl.ds((b * 3 + r) * AW, AW)], sem))
        copies.append(pltpu.make_async_copy(
            st_hbm.at[pl.ds(b * HA + start, AW)],
            st_v.at[pl.ds(b * AW, AW)], sem))
    for cp in copies:
        cp.start()
    for cp in copies:
        cp.wait()

    iota = lax.iota(jnp.int32, LANES)
    dup_lo = jnp.where(wid == NW - 1, LAST_DUP_STEPS, 0)

    for b in range(B):
        def body(s, carry, b=b):
            cls_acc, xy_acc, ang_acc, np_acc = carry
            base = s * LANES
            rows = iota + base
            keepf = jnp.where(s >= dup_lo, jnp.float32(1.0), jnp.float32(0.0))
            keepv = jnp.full((LANES,), 1.0, jnp.float32) * keepf

            ax = plsc.load_gather(anch_v, [rows])
            ay = plsc.load_gather(anch_v.at[pl.ds(AW, AW)], [rows])
            aa = plsc.load_gather(anch_v.at[pl.ds(2 * AW, AW)], [rows])

            abase = b * M * 2 * LANES
            gx = annb_v[pl.ds(abase, LANES)]
            gy = annb_v[pl.ds(abase + LANES, LANES)]
            dx = ax - gx
            dy = ay - gy
            best = dx * dx + dy * dy
            bidx = jnp.zeros((LANES,), jnp.int32)
            for m in range(1, M):
                gx = annb_v[pl.ds(abase + m * 2 * LANES, LANES)]
                gy = annb_v[pl.ds(abase + m * 2 * LANES + LANES, LANES)]
                dx = ax - gx
                dy = ay - gy
                sq = dx * dx + dy * dy
                upd = sq < best
                best = jnp.where(upd, sq, best)
                bidx = jnp.where(upd, _splat_i32(m), bidx)

            tbase = b * 4 * M
            gxb = plsc.load_gather(annt_v, [bidx + tbase])
            gyb = plsc.load_gather(annt_v, [bidx + (tbase + M)])
            gab = plsc.load_gather(annt_v, [bidx + (tbase + 2 * M)])
            gcb = plsc.load_gather(annt_v, [bidx + (tbase + 3 * M)])

            a = jnp.abs(aa - gab)
            negm = jnp.logical_or(best >= T_NEG, a >= NEG_ANGLE_DIST)
            pos = jnp.logical_and(best < T_POS, a < MAX_ANGLE_DIST)
            valid = jnp.logical_or(negm, pos)

            st = plsc.load_gather(st_v.at[pl.ds(b * AW, AW)], [rows])
            damp = jnp.where(st > jnp.float32(0.5), keepv, DAMPENING * keepv)
            dampv = jnp.where(valid, damp, jnp.float32(0.0))
            posf = jnp.where(pos, keepv, jnp.float32(0.0))

            step_sum = jnp.zeros((LANES,), jnp.float32)
            for c in range(C):
                clsv = plsc.load_gather(
                    cls_v.at[pl.ds((b * C + c) * AW, AW)], [rows])
                clsv = jnp.clip(clsv, jnp.float32(1e-4), jnp.float32(1.0 - 1e-4))
                is_one = jnp.logical_and(pos, gcb == jnp.float32(c))
                v = jnp.where(is_one, clsv, jnp.float32(1.0) - clsv)
                af = jnp.where(is_one, ALPHA, ONE_MINUS_ALPHA)
                fb = jnp.float32(1.0) - v
                lnv = _soft_log(v)
                step_sum = step_sum + (af * fb * fb) * lnv
            cls_acc = cls_acc - dampv * step_sum

            rx = plsc.load_gather(reg_v.at[pl.ds((b * 3 + 0) * AW, AW)], [rows])
            ry = plsc.load_gather(reg_v.at[pl.ds((b * 3 + 1) * AW, AW)], [rows])
            ra = plsc.load_gather(reg_v.at[pl.ds((b * 3 + 2) * AW, AW)], [rows])
            pdx = (ax + rx) - gxb
            pdy = (ay + ry) - gyb
            xy = _soft_sqrt(pdx * pdx + pdy * pdy + jnp.float32(1e-12))
            ang = jnp.abs((aa + ra) - gab)
            xy_acc = xy_acc + posf * _smooth_l1(xy)
            ang_acc = ang_acc + posf * _smooth_l1(ang)
            np_acc = np_acc + posf
            return cls_acc, xy_acc, ang_acc, np_acc

        zero = jnp.zeros((LANES,), jnp.float32)
        cls_acc, xy_acc, ang_acc, np_acc = lax.fori_loop(
            0, STEPS, body, (zero, zero, zero, zero))
        out_v[pl.ds((b * 4 + 0) * LANES, LANES)] = cls_acc
        out_v[pl.ds((b * 4 + 1) * LANES, LANES)] = xy_acc
        out_v[pl.ds((b * 4 + 2) * LANES, LANES)] = ang_acc
        out_v[pl.ds((b * 4 + 3) * LANES, LANES)] = np_acc

    pltpu.sync_copy(out_v, out_hbm.at[pl.ds(wid * B * 4 * LANES, B * 4 * LANES)])


def kernel(classifications, regressions, anchors, annotations, states,
           img_paths):
    one = jnp.float32(1.0) + jnp.float32(0.0) * img_paths[0].astype(jnp.float32)
    annt = (jnp.transpose(annotations, (0, 2, 1)) * one).reshape(-1)
    annb = (jnp.broadcast_to(annotations[:, :, :2, None],
                             (B, M, 2, LANES)) * one).reshape(-1)

    mesh = plsc.VectorSubcoreMesh(core_axis_name="c", subcore_axis_name="s",
                                  num_cores=2, num_subcores=16)
    run = pl.kernel(
        _focal_body,
        out_type=jax.ShapeDtypeStruct((NW * B * 4 * LANES,), jnp.float32),
        mesh=mesh,
        compiler_params=pltpu.CompilerParams(needs_layout_passes=False),
        scratch_types=[
            pltpu.VMEM((B * C * AW,), jnp.float32),
            pltpu.VMEM((B * 3 * AW,), jnp.float32),
            pltpu.VMEM((B * AW,), jnp.float32),
            pltpu.VMEM((3 * AW,), jnp.float32),
            pltpu.VMEM((B * M * 2 * LANES,), jnp.float32),
            pltpu.VMEM((B * 4 * M,), jnp.float32),
            pltpu.VMEM((B * 4 * LANES,), jnp.float32),
            pltpu.SemaphoreType.DMA,
        ],
    )
    part_list = []
    for h in range(2):
        sl = slice(h * HA, (h + 1) * HA)
        clsf = classifications[:, sl].transpose(0, 2, 1).reshape(-1) * one
        regf = regressions[:, sl].transpose(0, 2, 1).reshape(-1) * one
        anchf = anchors[0, sl].T.reshape(-1) * one
        stf = states[:, sl].reshape(-1) * one
        part_list.append(run(clsf, regf, anchf, annb, annt, stf))
    parts = part_list[0] + part_list[1]
    sums = jnp.sum(parts.reshape(NW, B, 4, LANES), axis=(0, 3))
    den = jnp.maximum(sums[:, 3], 1.0)
    cls_t = sums[:, 0] / den
    xy_t = sums[:, 1] / den
    ang_t = sums[:, 2] / den
    return jnp.stack([jnp.mean(cls_t), jnp.mean(xy_t), jnp.mean(ang_t)])


# speedup vs baseline: 1.0465x; 1.0465x over previous
import numpy as np
import jax
import jax.numpy as jnp
from jax import lax
from jax.experimental import pallas as pl
from jax.experimental.pallas import tpu as pltpu
from jax.experimental.pallas import tpu_sc as plsc

B, A, C, M = 4, 100000, 4, 32
NW = 32
LANES = 16
AW = 3200
STEPS = AW // LANES
LAST_START = A - AW
LAST_DUP_STEPS = (30 * AW + AW - LAST_START) // LANES

ALPHA = np.float32(0.95)
ONE_MINUS_ALPHA = np.float32(0.05)
DAMPENING = np.float32(0.5)
T_NEG = np.float32(900.0)
T_POS = np.float32(399.99996948242188)
MAX_ANGLE_DIST = np.float32(30.0)
NEG_ANGLE_DIST = np.float32(45.0)
LN2 = np.float32(0.6931471805599453)


def _splat_i32(v):
    return jnp.full((LANES,), v, jnp.int32)


def _soft_log(v):
    bits = lax.bitcast_convert_type(v, jnp.int32)
    e = (bits >> 23) - 127
    mbits = (bits & jnp.int32(0x7FFFFF)) | jnp.int32(0x3F800000)
    m = lax.bitcast_convert_type(mbits, jnp.float32)
    ef = e.astype(jnp.float32)
    big = m > jnp.float32(1.4142135)
    m = jnp.where(big, m * jnp.float32(0.5), m)
    ef = jnp.where(big, ef + jnp.float32(1.0), ef)
    t = (m - jnp.float32(1.0)) / (m + jnp.float32(1.0))
    t2 = t * t
    p = jnp.float32(1.0 / 7.0)
    p = p * t2 + jnp.float32(1.0 / 5.0)
    p = p * t2 + jnp.float32(1.0 / 3.0)
    p = p * t2 + jnp.float32(1.0)
    return ef * LN2 + jnp.float32(2.0) * t * p


def _soft_sqrt(x):
    bits = lax.bitcast_convert_type(x, jnp.int32)
    y = lax.bitcast_convert_type(jnp.int32(0x5F3759DF) - (bits >> 1), jnp.float32)
    xh = jnp.float32(0.5) * x
    for _ in range(3):
        y = y * (jnp.float32(1.5) - xh * y * y)
    return x * y


def _smooth_l1(d):
    return jnp.where(d <= jnp.float32(1.0 / 9.0),
                     jnp.float32(4.5) * d * d,
                     d - jnp.float32(1.0 / 18.0))


def _focal_body(cls_hbm, reg_hbm, anch_hbm, annb_hbm, annt_hbm, st_hbm,
                out_hbm, cls_v, reg_v, st_v, anch_v, annb_v, annt_v, out_v,
                sem):
    wid = lax.axis_index("s") * 2 + lax.axis_index("c")
    start = jnp.minimum(wid * AW, LAST_START)

    copies = [
        pltpu.make_async_copy(annb_hbm, annb_v, sem),
        pltpu.make_async_copy(annt_hbm, annt_v, sem),
    ]
    for k in range(3):
        copies.append(pltpu.make_async_copy(
            anch_hbm.at[pl.ds(k * A + start, AW)],
            anch_v.at[pl.ds(k * AW, AW)], sem))
    for b in range(B):
        for c in range(C):
            copies.append(pltpu.make_async_copy(
                cls_hbm.at[pl.ds((b * C + c) * A + start, AW)],
                cls_v.at[pl.ds((b * C + c) * AW, AW)], sem))
        for r in range(3):
            copies.append(pltpu.make_async_copy(
                reg_hbm.at[pl.ds((b * 3 + r) * A + start, AW)],
                reg_v.at[pl.ds((b * 3 + r) * AW, AW)], sem))
        copies.append(pltpu.make_async_copy(
            st_hbm.at[pl.ds(b * A + start, AW)],
            st_v.at[pl.ds(b * AW, AW)], sem))
    for cp in copies:
        cp.start()
    for cp in copies:
        cp.wait()

    iota = lax.iota(jnp.int32, LANES)
    dup_lo = jnp.where(wid == NW - 1, LAST_DUP_STEPS, 0)

    for b in range(B):
        def body(s, carry, b=b):
            cls_acc, xy_acc, ang_acc, np_acc = carry
            base = s * LANES
            rows = iota + base
            keepf = jnp.where(s >= dup_lo, jnp.float32(1.0), jnp.float32(0.0))
            keepv = jnp.full((LANES,), 1.0, jnp.float32) * keepf

            ax = plsc.load_gather(anch_v, [rows])
            ay = plsc.load_gather(anch_v, [rows + AW])
            aa = plsc.load_gather(anch_v, [rows + 2 * AW])

            abase = b * M * 2 * LANES
            gx = annb_v[pl.ds(abase, LANES)]
            gy = annb_v[pl.ds(abase + LANES, LANES)]
            dx = ax - gx
            dy = ay - gy
            best = dx * dx + dy * dy
            bidx = jnp.zeros((LANES,), jnp.int32)
            for m in range(1, M):
                gx = annb_v[pl.ds(abase + m * 2 * LANES, LANES)]
                gy = annb_v[pl.ds(abase + m * 2 * LANES + LANES, LANES)]
                dx = ax - gx
                dy = ay - gy
                sq = dx * dx + dy * dy
                upd = sq < best
                best = jnp.where(upd, sq, best)
                bidx = jnp.where(upd, _splat_i32(m), bidx)

            tbase = b * 4 * M
            gxb = plsc.load_gather(annt_v, [bidx + tbase])
            gyb = plsc.load_gather(annt_v, [bidx + (tbase + M)])
            gab = plsc.load_gather(annt_v, [bidx + (tbase + 2 * M)])
            gcb = plsc.load_gather(annt_v, [bidx + (tbase + 3 * M)])

            a = jnp.abs(aa - gab)
            negm = jnp.logical_or(best >= T_NEG, a >= NEG_ANGLE_DIST)
            pos = jnp.logical_and(best < T_POS, a < MAX_ANGLE_DIST)
            valid = jnp.logical_or(negm, pos)

            st = plsc.load_gather(st_v, [rows + b * AW])
            damp = jnp.where(st > jnp.float32(0.5), keepv, DAMPENING * keepv)
            dampv = jnp.where(valid, damp, jnp.float32(0.0))
            posf = jnp.where(pos, keepv, jnp.float32(0.0))

            step_sum = jnp.zeros((LANES,), jnp.float32)
            for c in range(C):
                clsv = plsc.load_gather(cls_v, [rows + (b * C + c) * AW])
                clsv = jnp.clip(clsv, jnp.float32(1e-4), jnp.float32(1.0 - 1e-4))
                is_one = jnp.logical_and(pos, gcb == jnp.float32(c))
                v = jnp.where(is_one, clsv, jnp.float32(1.0) - clsv)
                af = jnp.where(is_one, ALPHA, ONE_MINUS_ALPHA)
                fb = jnp.float32(1.0) - v
                lnv = _soft_log(v)
                step_sum = step_sum + (af * fb * fb) * lnv
            cls_acc = cls_acc - dampv * step_sum

            rx = plsc.load_gather(reg_v, [rows + (b * 3 + 0) * AW])
            ry = plsc.load_gather(reg_v, [rows + (b * 3 + 1) * AW])
            ra = plsc.load_gather(reg_v, [rows + (b * 3 + 2) * AW])
            pdx = (ax + rx) - gxb
            pdy = (ay + ry) - gyb
            xy = _soft_sqrt(pdx * pdx + pdy * pdy + jnp.float32(1e-12))
            ang = jnp.abs((aa + ra) - gab)
            xy_acc = xy_acc + posf * _smooth_l1(xy)
            ang_acc = ang_acc + posf * _smooth_l1(ang)
            np_acc = np_acc + posf
            return cls_acc, xy_acc, ang_acc, np_acc

        zero = jnp.zeros((LANES,), jnp.float32)
        cls_acc, xy_acc, ang_acc, np_acc = lax.fori_loop(
            0, STEPS, body, (zero, zero, zero, zero))
        out_v[pl.ds((b * 4 + 0) * LANES, LANES)] = cls_acc
        out_v[pl.ds((b * 4 + 1) * LANES, LANES)] = xy_acc
        out_v[pl.ds((b * 4 + 2) * LANES, LANES)] = ang_acc
        out_v[pl.ds((b * 4 + 3) * LANES, LANES)] = np_acc

    pltpu.sync_copy(out_v, out_hbm.at[pl.ds(wid * B * 4 * LANES, B * 4 * LANES)])


def kernel(classifications, regressions, anchors, annotations, states,
           img_paths):
    one = jnp.float32(1.0) + jnp.float32(0.0) * img_paths[0].astype(jnp.float32)
    clsf = classifications.transpose(0, 2, 1).reshape(-1) * one
    regf = regressions.transpose(0, 2, 1).reshape(-1) * one
    anchf = anchors[0].T.reshape(-1) * one
    stf = states.reshape(-1) * one
    annt = (jnp.transpose(annotations, (0, 2, 1)) * one).reshape(-1)
    annb = (jnp.broadcast_to(annotations[:, :, :2, None],
                             (B, M, 2, LANES)) * one).reshape(-1)

    mesh = plsc.VectorSubcoreMesh(core_axis_name="c", subcore_axis_name="s",
                                  num_cores=2, num_subcores=16)
    run = pl.kernel(
        _focal_body,
        out_type=jax.ShapeDtypeStruct((NW * B * 4 * LANES,), jnp.float32),
        mesh=mesh,
        compiler_params=pltpu.CompilerParams(needs_layout_passes=False),
        scratch_types=[
            pltpu.VMEM((B * C * AW,), jnp.float32),
            pltpu.VMEM((B * 3 * AW,), jnp.float32),
            pltpu.VMEM((B * AW,), jnp.float32),
            pltpu.VMEM((3 * AW,), jnp.float32),
            pltpu.VMEM((B * M * 2 * LANES,), jnp.float32),
            pltpu.VMEM((B * 4 * M,), jnp.float32),
            pltpu.VMEM((B * 4 * LANES,), jnp.float32),
            pltpu.SemaphoreType.DMA,
        ],
    )
    parts = run(clsf, regf, anchf, annb, annt, stf)
    sums = jnp.sum(parts.reshape(NW, B, 4, LANES), axis=(0, 3))
    den = jnp.maximum(sums[:, 3], 1.0)
    cls_t = sums[:, 0] / den
    xy_t = sums[:, 1] / den
    ang_t = sums[:, 2] / den
    return jnp.stack([jnp.mean(cls_t), jnp.mean(xy_t), jnp.mean(ang_t)])


# speedup vs baseline: 1.1054x; 1.0562x over previous
import numpy as np
import jax
import jax.numpy as jnp
from jax import lax
from jax.experimental import pallas as pl
from jax.experimental.pallas import tpu as pltpu
from jax.experimental.pallas import tpu_sc as plsc

B, A, C, M = 4, 100000, 4, 32
NW = 32
LANES = 16
AW = 3136
STEPS = AW // LANES
LAST_START = A - AW
LAST_DUP_STEPS = (30 * AW + AW - LAST_START) // LANES

ALPHA = np.float32(0.95)
ONE_MINUS_ALPHA = np.float32(0.05)
DAMPENING = np.float32(0.5)
T_NEG = np.float32(900.0)
T_POS = np.float32(399.99996948242188)
MAX_ANGLE_DIST = np.float32(30.0)
NEG_ANGLE_DIST = np.float32(45.0)
LN2 = np.float32(0.6931471805599453)


def _splat_i32(v):
    return jnp.full((LANES,), v, jnp.int32)


def _soft_log(v):
    bits = lax.bitcast_convert_type(v, jnp.int32)
    e = (bits >> 23) - 127
    mbits = (bits & jnp.int32(0x7FFFFF)) | jnp.int32(0x3F800000)
    m = lax.bitcast_convert_type(mbits, jnp.float32)
    ef = e.astype(jnp.float32)
    big = m > jnp.float32(1.4142135)
    m = jnp.where(big, m * jnp.float32(0.5), m)
    ef = jnp.where(big, ef + jnp.float32(1.0), ef)
    t = (m - jnp.float32(1.0)) / (m + jnp.float32(1.0))
    t2 = t * t
    p = jnp.float32(1.0 / 7.0)
    p = p * t2 + jnp.float32(1.0 / 5.0)
    p = p * t2 + jnp.float32(1.0 / 3.0)
    p = p * t2 + jnp.float32(1.0)
    return ef * LN2 + jnp.float32(2.0) * t * p


def _soft_sqrt(x):
    bits = lax.bitcast_convert_type(x, jnp.int32)
    y = lax.bitcast_convert_type(jnp.int32(0x5F3759DF) - (bits >> 1), jnp.float32)
    xh = jnp.float32(0.5) * x
    for _ in range(3):
        y = y * (jnp.float32(1.5) - xh * y * y)
    return x * y


def _smooth_l1(d):
    return jnp.where(d <= jnp.float32(1.0 / 9.0),
                     jnp.float32(4.5) * d * d,
                     d - jnp.float32(1.0 / 18.0))


def _focal_body(cls_hbm, reg_hbm, anch_hbm, annb_hbm, annt_hbm, st_hbm,
                out_hbm, cls_v, reg_v, st_v, anch_v, annb_v, annt_v, out_v,
                sem):
    wid = lax.axis_index("s") * 2 + lax.axis_index("c")
    start = jnp.minimum(wid * AW, LAST_START)

    copies = [
        pltpu.make_async_copy(annb_hbm, annb_v, sem),
        pltpu.make_async_copy(annt_hbm, annt_v, sem),
    ]
    for k in range(3):
        copies.append(pltpu.make_async_copy(
            anch_hbm.at[pl.ds(k * A + start, AW)],
            anch_v.at[pl.ds(k * AW, AW)], sem))
    for b in range(B):
        for c in range(C):
            copies.append(pltpu.make_async_copy(
                cls_hbm.at[pl.ds((b * C + c) * A + start, AW)],
                cls_v.at[pl.ds((b * C + c) * AW, AW)], sem))
        for r in range(3):
            copies.append(pltpu.make_async_copy(
                reg_hbm.at[pl.ds((b * 3 + r) * A + start, AW)],
                reg_v.at[pl.ds((b * 3 + r) * AW, AW)], sem))
        copies.append(pltpu.make_async_copy(
            st_hbm.at[pl.ds(b * A + start, AW)],
            st_v.at[pl.ds(b * AW, AW)], sem))
    for cp in copies:
        cp.start()
    for cp in copies:
        cp.wait()

    iota = lax.iota(jnp.int32, LANES)
    dup_lo = jnp.where(wid == NW - 1, LAST_DUP_STEPS, 0)

    for b in range(B):
        def body(s, carry, b=b):
            cls_acc, xy_acc, ang_acc, np_acc = carry
            base = s * LANES
            rows = iota + base
            keepf = jnp.where(s >= dup_lo, jnp.float32(1.0), jnp.float32(0.0))
            keepv = jnp.full((LANES,), 1.0, jnp.float32) * keepf

            ax = plsc.load_gather(anch_v, [rows])
            ay = plsc.load_gather(anch_v.at[pl.ds(AW, AW)], [rows])
            aa = plsc.load_gather(anch_v.at[pl.ds(2 * AW, AW)], [rows])

            abase = b * M * 2 * LANES
            gx = annb_v[pl.ds(abase, LANES)]
            gy = annb_v[pl.ds(abase + LANES, LANES)]
            dx = ax - gx
            dy = ay - gy
            best = dx * dx + dy * dy
            bidx = jnp.zeros((LANES,), jnp.int32)
            for m in range(1, M):
                gx = annb_v[pl.ds(abase + m * 2 * LANES, LANES)]
                gy = annb_v[pl.ds(abase + m * 2 * LANES + LANES, LANES)]
                dx = ax - gx
                dy = ay - gy
                sq = dx * dx + dy * dy
                upd = sq < best
                best = jnp.where(upd, sq, best)
                bidx = jnp.where(upd, _splat_i32(m), bidx)

            tbase = b * 4 * M
            gxb = plsc.load_gather(annt_v, [bidx + tbase])
            gyb = plsc.load_gather(annt_v, [bidx + (tbase + M)])
            gab = plsc.load_gather(annt_v, [bidx + (tbase + 2 * M)])
            gcb = plsc.load_gather(annt_v, [bidx + (tbase + 3 * M)])

            a = jnp.abs(aa - gab)
            negm = jnp.logical_or(best >= T_NEG, a >= NEG_ANGLE_DIST)
            pos = jnp.logical_and(best < T_POS, a < MAX_ANGLE_DIST)
            valid = jnp.logical_or(negm, pos)

            st = plsc.load_gather(st_v.at[pl.ds(b * AW, AW)], [rows])
            damp = jnp.where(st > jnp.float32(0.5), keepv, DAMPENING * keepv)
            dampv = jnp.where(valid, damp, jnp.float32(0.0))
            posf = jnp.where(pos, keepv, jnp.float32(0.0))

            step_sum = jnp.zeros((LANES,), jnp.float32)
            for c in range(C):
                clsv = plsc.load_gather(
                    cls_v.at[pl.ds((b * C + c) * AW, AW)], [rows])
                clsv = jnp.clip(clsv, jnp.float32(1e-4), jnp.float32(1.0 - 1e-4))
                is_one = jnp.logical_and(pos, gcb == jnp.float32(c))
                v = jnp.where(is_one, clsv, jnp.float32(1.0) - clsv)
                af = jnp.where(is_one, ALPHA, ONE_MINUS_ALPHA)
                fb = jnp.float32(1.0) - v
                lnv = _soft_log(v)
                step_sum = step_sum + (af * fb * fb) * lnv
            cls_acc = cls_acc - dampv * step_sum

            rx = plsc.load_gather(reg_v.at[pl.ds((b * 3 + 0) * AW, AW)], [rows])
            ry = plsc.load_gather(reg_v.at[pl.ds((b * 3 + 1) * AW, AW)], [rows])
            ra = plsc.load_gather(reg_v.at[pl.ds((b * 3 + 2) * AW, AW)], [rows])
            pdx = (ax + rx) - gxb
            pdy = (ay + ry) - gyb
            xy = _soft_sqrt(pdx * pdx + pdy * pdy + jnp.float32(1e-12))
            ang = jnp.abs((aa + ra) - gab)
            xy_acc = xy_acc + posf * _smooth_l1(xy)
            ang_acc = ang_acc + posf * _smooth_l1(ang)
            np_acc = np_acc + posf
            return cls_acc, xy_acc, ang_acc, np_acc

        zero = jnp.zeros((LANES,), jnp.float32)
        cls_acc, xy_acc, ang_acc, np_acc = lax.fori_loop(
            0, STEPS, body, (zero, zero, zero, zero))
        out_v[pl.ds((b * 4 + 0) * LANES, LANES)] = cls_acc
        out_v[pl.ds((b * 4 + 1) * LANES, LANES)] = xy_acc
        out_v[pl.ds((b * 4 + 2) * LANES, LANES)] = ang_acc
        out_v[pl.ds((b * 4 + 3) * LANES, LANES)] = np_acc

    pltpu.sync_copy(out_v, out_hbm.at[pl.ds(wid * B * 4 * LANES, B * 4 * LANES)])


def kernel(classifications, regressions, anchors, annotations, states,
           img_paths):
    one = jnp.float32(1.0) + jnp.float32(0.0) * img_paths[0].astype(jnp.float32)
    clsf = classifications.transpose(0, 2, 1).reshape(-1) * one
    regf = regressions.transpose(0, 2, 1).reshape(-1) * one
    anchf = anchors[0].T.reshape(-1) * one
    stf = states.reshape(-1) * one
    annt = (jnp.transpose(annotations, (0, 2, 1)) * one).reshape(-1)
    annb = (jnp.broadcast_to(annotations[:, :, :2, None],
                             (B, M, 2, LANES)) * one).reshape(-1)

    mesh = plsc.VectorSubcoreMesh(core_axis_name="c", subcore_axis_name="s",
                                  num_cores=2, num_subcores=16)
    run = pl.kernel(
        _focal_body,
        out_type=jax.ShapeDtypeStruct((NW * B * 4 * LANES,), jnp.float32),
        mesh=mesh,
        compiler_params=pltpu.CompilerParams(needs_layout_passes=False),
        scratch_types=[
            pltpu.VMEM((B * C * AW,), jnp.float32),
            pltpu.VMEM((B * 3 * AW,), jnp.float32),
            pltpu.VMEM((B * AW,), jnp.float32),
            pltpu.VMEM((3 * AW,), jnp.float32),
            pltpu.VMEM((B * M * 2 * LANES,), jnp.float32),
            pltpu.VMEM((B * 4 * M,), jnp.float32),
            pltpu.VMEM((B * 4 * LANES,), jnp.float32),
            pltpu.SemaphoreType.DMA,
        ],
    )
    parts = run(clsf, regf, anchf, annb, annt, stf)
    sums = jnp.sum(parts.reshape(NW, B, 4, LANES), axis=(0, 3))
    den = jnp.maximum(sums[:, 3], 1.0)
    cls_t = sums[:, 0] / den
    xy_t = sums[:, 1] / den
    ang_t = sums[:, 2] / den
    return jnp.stack([jnp.mean(cls_t), jnp.mean(xy_t), jnp.mean(ang_t)])
